# baseline (speedup 1.0000x reference)
import functools

import jax
import jax.numpy as jnp
from jax import lax
from jax.experimental import pallas as pl
from jax.experimental.pallas import tpu as pltpu
from jax.experimental.pallas import tpu_sc as plsc

B = 16384
SKILL_LEN = 50
SKILL_VOCAB = 10000
FEAT = 192
NC, NS = 2, 16
NW = NC * NS
BPW = B // NW
SPC = 8
SK_CHUNK = SPC * SKILL_LEN
NCHUNK = BPW // SPC
VROWS = SKILL_VOCAB // NS


def _sc_gather_body(job_hbm, cat_hbm, loc_hbm, lev_hbm, tok_hbm,
                    job_t, cat_t, loc_t, lev_t, skill_t,
                    feat_hbm,
                    idxa, idxb, r32a, r32b, r64,
                    tok0, tok1, sr0, sr1, out_v,
                    skill_sh, gsem0, gsem1, wsem):
    sid = lax.axis_index("s")
    wid = sid * NC + lax.axis_index("c")
    base = wid * BPW
    rows = pl.ds(base, BPW)

    pltpu.sync_copy(skill_t.at[pl.ds(sid * VROWS, VROWS), :],
                    skill_sh.at[pl.ds(sid * VROWS, VROWS), :])

    pltpu.sync_copy(job_hbm.at[rows], idxa)
    g_job = pltpu.async_copy(job_t.at[idxa], r32a, gsem0)
    pltpu.sync_copy(loc_hbm.at[rows], idxb)
    g_loc = pltpu.async_copy(loc_t.at[idxb], r32b, gsem1)
    g_job.wait()
    w_job = pltpu.async_copy(r32a, feat_hbm.at[rows, pl.ds(0, 32)], wsem)
    g_loc.wait()
    w_loc = pltpu.async_copy(r32b, feat_hbm.at[rows, pl.ds(96, 32)], wsem)
    pltpu.sync_copy(cat_hbm.at[rows], idxa)
    g_cat = pltpu.async_copy(cat_t.at[idxa], r64, gsem0)
    pltpu.sync_copy(lev_hbm.at[rows], idxb)
    w_job.wait()
    g_lev = pltpu.async_copy(lev_t.at[idxb], r32a, gsem1)
    g_cat.wait()
    w_cat = pltpu.async_copy(r64, feat_hbm.at[rows, pl.ds(32, 64)], wsem)
    g_lev.wait()
    w_lev = pltpu.async_copy(r32a, feat_hbm.at[rows, pl.ds(128, 32)], wsem)

    plsc.subcore_barrier()

    def load_tok(c, tok_v):
        pltpu.sync_copy(tok_hbm.at[pl.ds(base + c * SPC, SPC), :], tok_v)

    def fire_gathers(tok_v, sr_v, sem):
        for s in range(SPC):
            pltpu.async_copy(skill_sh.at[tok_v.at[s]],
                             sr_v.at[pl.ds(s * SKILL_LEN, SKILL_LEN)], sem)

    def wait_gathers(tok_v, sr_v, sem):
        for s in range(SPC):
            pltpu.make_async_copy(
                skill_sh.at[tok_v.at[s]],
                sr_v.at[pl.ds(s * SKILL_LEN, SKILL_LEN)], sem).wait()

    def reduce_chunk(c, sr_v):
        grp = 10
        for s in range(SPC):
            r0 = s * SKILL_LEN
            row = c * SPC + s
            for g in range(SKILL_LEN // grp):
                b0 = r0 + g * grp
                a0 = sr_v[b0, pl.ds(0, 16)]
                a1 = sr_v[b0, pl.ds(16, 16)]
                for j in range(1, grp):
                    a0 = a0 + sr_v[b0 + j, pl.ds(0, 16)]
                    a1 = a1 + sr_v[b0 + j, pl.ds(16, 16)]
                if g == 0:
                    out_v[row, pl.ds(0, 16)] = a0
                    out_v[row, pl.ds(16, 16)] = a1
                else:
                    plsc.addupdate(out_v.at[row, pl.ds(0, 16)], a0)
                    plsc.addupdate(out_v.at[row, pl.ds(16, 16)], a1)

    load_tok(0, tok0)
    fire_gathers(tok0, sr0, gsem0)
    load_tok(1, tok1)
    fire_gathers(tok1, sr1, gsem1)

    @pl.loop(0, NCHUNK - 2, step=2)
    def _(c):
        wait_gathers(tok0, sr0, gsem0)
        reduce_chunk(c, sr0)
        load_tok(c + 2, tok0)
        fire_gathers(tok0, sr0, gsem0)
        wait_gathers(tok1, sr1, gsem1)
        reduce_chunk(c + 1, sr1)
        load_tok(c + 3, tok1)
        fire_gathers(tok1, sr1, gsem1)

    wait_gathers(tok0, sr0, gsem0)
    reduce_chunk(NCHUNK - 2, sr0)
    wait_gathers(tok1, sr1, gsem1)
    reduce_chunk(NCHUNK - 1, sr1)

    pltpu.sync_copy(out_v, feat_hbm.at[rows, pl.ds(160, 32)])
    w_loc.wait()
    w_cat.wait()
    w_lev.wait()


@jax.jit
def _sc_gather(job_id, category, location, level, tok2d,
               job_t, cat_t, loc_t, lev_t, skill_t):
    f32 = jnp.float32
    i32 = jnp.int32
    out_type = jax.ShapeDtypeStruct((B, FEAT), f32)
    scratch = [
        pltpu.VMEM((BPW,), i32),
        pltpu.VMEM((BPW,), i32),
        pltpu.VMEM((BPW, 32), f32),
        pltpu.VMEM((BPW, 32), f32),
        pltpu.VMEM((BPW, 64), f32),
        pltpu.VMEM((SPC, SKILL_LEN), i32),
        pltpu.VMEM((SPC, SKILL_LEN), i32),
        pltpu.VMEM((SK_CHUNK, 32), f32),
        pltpu.VMEM((SK_CHUNK, 32), f32),
        pltpu.VMEM((BPW, 32), f32),
        pltpu.VMEM_SHARED((SKILL_VOCAB, 32), f32),
        pltpu.SemaphoreType.DMA,
        pltpu.SemaphoreType.DMA,
        pltpu.SemaphoreType.DMA,
    ]
    mesh = plsc.VectorSubcoreMesh(core_axis_name="c", subcore_axis_name="s")
    k = pl.kernel(_sc_gather_body, out_type=out_type, mesh=mesh,
                  scratch_types=scratch,
                  compiler_params=pltpu.CompilerParams(
                      use_tc_tiling_on_sc=False))
    return k(job_id, category, location, level, tok2d,
             job_t, cat_t, loc_t, lev_t, skill_t)


BB = 512


def _mlp_body(feat, W1, b1, W2, b2, W3, b3, out):
    h = jnp.maximum(jnp.dot(feat[...], W1[...],
                            preferred_element_type=jnp.float32) + b1[...], 0.0)
    h = jnp.maximum(jnp.dot(h, W2[...],
                            preferred_element_type=jnp.float32) + b2[...], 0.0)
    out[...] = jnp.dot(h, W3[...],
                       preferred_element_type=jnp.float32) + b3[...]


@jax.jit
def _tc_mlp(feat, W1, b1, W2, b2, W3, b3):
    nb = B // BB
    full = lambda a: pl.BlockSpec(a.shape, lambda i: tuple(0 for _ in a.shape))
    return pl.pallas_call(
        _mlp_body,
        grid=(nb,),
        in_specs=[pl.BlockSpec((BB, FEAT), lambda i: (i, 0)),
                  full(W1), full(b1), full(W2), full(b2), full(W3), full(b3)],
        out_specs=pl.BlockSpec((BB, 64), lambda i: (i, 0)),
        out_shape=jax.ShapeDtypeStruct((B, 64), jnp.float32),
    )(feat, W1, b1, W2, b2, W3, b3)


def kernel(job_id, category, location, level, skill_tokens,
           job_table, category_table, location_table, level_table, skill_table,
           W1, b1, W2, b2, W3, b3):
    feat = _sc_gather(job_id, category, location, level, skill_tokens,
                      job_table, category_table, location_table,
                      level_table, skill_table)
    scale = jnp.concatenate([jnp.ones((160,), jnp.float32),
                             jnp.full((32,), 1.0 / SKILL_LEN, jnp.float32)])
    W1s = W1 * scale[:, None]
    return _tc_mlp(feat, W1s, b1, W2, b2, W3, b3)


# speedup vs baseline: 1.1602x; 1.1602x over previous
import functools

import jax
import jax.numpy as jnp
from jax import lax
from jax.experimental import pallas as pl
from jax.experimental.pallas import tpu as pltpu
from jax.experimental.pallas import tpu_sc as plsc

B = 16384
SKILL_LEN = 50
FEAT = 192
NC, NS = 2, 16
NW = NC * NS
BPW = B // NW
SPC = 8
SK_CHUNK = SPC * SKILL_LEN
NCHUNK = BPW // SPC


def _sc_gather_body(job_hbm, cat_hbm, loc_hbm, lev_hbm, tok_hbm,
                    job_t, cat_t, loc_t, lev_t, skill_t,
                    feat_hbm,
                    idxa, idxb, r32a, r32b, r64,
                    tok0, tok1, sr0, sr1, out_v,
                    gsem0, gsem1, lsem0, lsem1, wsem):
    sid = lax.axis_index("s")
    wid = sid * NC + lax.axis_index("c")
    base = wid * BPW
    rows = pl.ds(base, BPW)

    pltpu.sync_copy(job_hbm.at[rows], idxa)
    g_job = pltpu.async_copy(job_t.at[idxa], r32a, gsem0)
    pltpu.sync_copy(loc_hbm.at[rows], idxb)
    g_loc = pltpu.async_copy(loc_t.at[idxb], r32b, gsem1)
    g_job.wait()
    w_job = pltpu.async_copy(r32a, feat_hbm.at[rows, pl.ds(0, 32)], wsem)
    g_loc.wait()
    w_loc = pltpu.async_copy(r32b, feat_hbm.at[rows, pl.ds(96, 32)], wsem)
    pltpu.sync_copy(cat_hbm.at[rows], idxa)
    g_cat = pltpu.async_copy(cat_t.at[idxa], r64, gsem0)
    pltpu.sync_copy(lev_hbm.at[rows], idxb)
    w_job.wait()
    g_lev = pltpu.async_copy(lev_t.at[idxb], r32a, gsem1)
    g_cat.wait()
    w_cat = pltpu.async_copy(r64, feat_hbm.at[rows, pl.ds(32, 64)], wsem)
    g_lev.wait()
    w_lev = pltpu.async_copy(r32a, feat_hbm.at[rows, pl.ds(128, 32)], wsem)

    sbase = wid * BPW * SKILL_LEN

    def load_tok(c, tok_v, sem):
        return pltpu.async_copy(
            tok_hbm.at[pl.ds(sbase + c * SK_CHUNK, SK_CHUNK)], tok_v, sem)

    def fire_gather(tok_v, sr_v, sem):
        return pltpu.async_copy(skill_t.at[tok_v], sr_v, sem)

    def wait_gather(tok_v, sr_v, sem):
        pltpu.make_async_copy(skill_t.at[tok_v], sr_v, sem).wait()

    def wait_tok(c, tok_v, sem):
        pltpu.make_async_copy(
            tok_hbm.at[pl.ds(sbase + c * SK_CHUNK, SK_CHUNK)], tok_v,
            sem).wait()

    def reduce_chunk(c, sr_v):
        grp = 10
        for s in range(SPC):
            r0 = s * SKILL_LEN
            row = c * SPC + s
            for g in range(SKILL_LEN // grp):
                b0 = r0 + g * grp
                a0 = sr_v[b0, pl.ds(0, 16)]
                a1 = sr_v[b0, pl.ds(16, 16)]
                for j in range(1, grp):
                    a0 = a0 + sr_v[b0 + j, pl.ds(0, 16)]
                    a1 = a1 + sr_v[b0 + j, pl.ds(16, 16)]
                if g == 0:
                    out_v[row, pl.ds(0, 16)] = a0
                    out_v[row, pl.ds(16, 16)] = a1
                else:
                    plsc.addupdate(out_v.at[row, pl.ds(0, 16)], a0)
                    plsc.addupdate(out_v.at[row, pl.ds(16, 16)], a1)

    load_tok(0, tok0, lsem0).wait()
    fire_gather(tok0, sr0, gsem0)
    load_tok(1, tok1, lsem1).wait()
    fire_gather(tok1, sr1, gsem1)

    @pl.loop(0, NCHUNK - 2, step=2)
    def _(c):
        wait_gather(tok0, sr0, gsem0)
        load_tok(c + 2, tok0, lsem0)
        reduce_chunk(c, sr0)
        wait_tok(c + 2, tok0, lsem0)
        fire_gather(tok0, sr0, gsem0)
        wait_gather(tok1, sr1, gsem1)
        load_tok(c + 3, tok1, lsem1)
        reduce_chunk(c + 1, sr1)
        wait_tok(c + 3, tok1, lsem1)
        fire_gather(tok1, sr1, gsem1)

    wait_gather(tok0, sr0, gsem0)
    reduce_chunk(NCHUNK - 2, sr0)
    wait_gather(tok1, sr1, gsem1)
    reduce_chunk(NCHUNK - 1, sr1)

    pltpu.sync_copy(out_v, feat_hbm.at[rows, pl.ds(160, 32)])
    w_loc.wait()
    w_cat.wait()
    w_lev.wait()


@jax.jit
def _sc_gather(job_id, category, location, level, tok_flat,
               job_t, cat_t, loc_t, lev_t, skill_t):
    f32 = jnp.float32
    i32 = jnp.int32
    out_type = jax.ShapeDtypeStruct((B, FEAT), f32)
    scratch = [
        pltpu.VMEM((BPW,), i32),
        pltpu.VMEM((BPW,), i32),
        pltpu.VMEM((BPW, 32), f32),
        pltpu.VMEM((BPW, 32), f32),
        pltpu.VMEM((BPW, 64), f32),
        pltpu.VMEM((SK_CHUNK,), i32),
        pltpu.VMEM((SK_CHUNK,), i32),
        pltpu.VMEM((SK_CHUNK, 32), f32),
        pltpu.VMEM((SK_CHUNK, 32), f32),
        pltpu.VMEM((BPW, 32), f32),
        pltpu.SemaphoreType.DMA,
        pltpu.SemaphoreType.DMA,
        pltpu.SemaphoreType.DMA,
        pltpu.SemaphoreType.DMA,
        pltpu.SemaphoreType.DMA,
    ]
    mesh = plsc.VectorSubcoreMesh(core_axis_name="c", subcore_axis_name="s")
    k = pl.kernel(_sc_gather_body, out_type=out_type, mesh=mesh,
                  scratch_types=scratch,
                  compiler_params=pltpu.CompilerParams(
                      use_tc_tiling_on_sc=False))
    return k(job_id, category, location, level, tok_flat,
             job_t, cat_t, loc_t, lev_t, skill_t)


BB = 512


def _mlp_body(feat, W1, b1, W2, b2, W3, b3, out):
    h = jnp.maximum(jnp.dot(feat[...], W1[...],
                            preferred_element_type=jnp.float32) + b1[...], 0.0)
    h = jnp.maximum(jnp.dot(h, W2[...],
                            preferred_element_type=jnp.float32) + b2[...], 0.0)
    out[...] = jnp.dot(h, W3[...],
                       preferred_element_type=jnp.float32) + b3[...]


@jax.jit
def _tc_mlp(feat, W1, b1, W2, b2, W3, b3):
    nb = B // BB
    full = lambda a: pl.BlockSpec(a.shape, lambda i: tuple(0 for _ in a.shape))
    return pl.pallas_call(
        _mlp_body,
        grid=(nb,),
        in_specs=[pl.BlockSpec((BB, FEAT), lambda i: (i, 0)),
                  full(W1), full(b1), full(W2), full(b2), full(W3), full(b3)],
        out_specs=pl.BlockSpec((BB, 64), lambda i: (i, 0)),
        out_shape=jax.ShapeDtypeStruct((B, 64), jnp.float32),
    )(feat, W1, b1, W2, b2, W3, b3)


def kernel(job_id, category, location, level, skill_tokens,
           job_table, category_table, location_table, level_table, skill_table,
           W1, b1, W2, b2, W3, b3):
    feat = _sc_gather(job_id, category, location, level,
                      skill_tokens.reshape(-1),
                      job_table, category_table, location_table,
                      level_table, skill_table)
    scale = jnp.concatenate([jnp.ones((160,), jnp.float32),
                             jnp.full((32,), 1.0 / SKILL_LEN, jnp.float32)])
    W1s = W1 * scale[:, None]
    return _tc_mlp(feat, W1s, b1, W2, b2, W3, b3)


# speedup vs baseline: 1.2074x; 1.0407x over previous
import functools

import jax
import jax.numpy as jnp
from jax import lax
from jax.experimental import pallas as pl
from jax.experimental.pallas import tpu as pltpu
from jax.experimental.pallas import tpu_sc as plsc

B = 16384
SKILL_LEN = 50
FEAT = 192
NC, NS = 2, 16
NW = NC * NS
BPW = B // NW
SPC = 8
SK_CHUNK = SPC * SKILL_LEN
NCHUNK = BPW // SPC
GRP = 10


def _sc_gather_body(job_hbm, cat_hbm, loc_hbm, lev_hbm, tok_hbm,
                    job_t, cat_t, loc_t, lev_t, skill_t,
                    feat_hbm,
                    idxa, idxb, r32a, r32b, r64,
                    tok0, tok1, sr0, sr1, out_v,
                    gsem0, gsem1, lsem0, lsem1, wsem):
    sid = lax.axis_index("s")
    wid = sid * NC + lax.axis_index("c")
    base = wid * BPW
    rows = pl.ds(base, BPW)

    pltpu.sync_copy(job_hbm.at[rows], idxa)
    g_job = pltpu.async_copy(job_t.at[idxa], r32a, gsem0)
    pltpu.sync_copy(loc_hbm.at[rows], idxb)
    g_loc = pltpu.async_copy(loc_t.at[idxb], r32b, gsem1)
    g_job.wait()
    w_job = pltpu.async_copy(r32a, feat_hbm.at[rows, pl.ds(0, 32)], wsem)
    g_loc.wait()
    w_loc = pltpu.async_copy(r32b, feat_hbm.at[rows, pl.ds(96, 32)], wsem)
    pltpu.sync_copy(cat_hbm.at[rows], idxa)
    g_cat = pltpu.async_copy(cat_t.at[idxa], r64, gsem0)
    pltpu.sync_copy(lev_hbm.at[rows], idxb)
    w_job.wait()
    g_lev = pltpu.async_copy(lev_t.at[idxb], r32a, gsem1)
    g_cat.wait()
    w_cat = pltpu.async_copy(r64, feat_hbm.at[rows, pl.ds(32, 64)], wsem)
    g_lev.wait()
    w_lev = pltpu.async_copy(r32a, feat_hbm.at[rows, pl.ds(128, 32)], wsem)

    sbase = wid * BPW * SKILL_LEN

    def load_tok(c, tok_v, sem):
        return pltpu.async_copy(
            tok_hbm.at[pl.ds(sbase + c * SK_CHUNK, SK_CHUNK)], tok_v, sem)

    def fire_gather(tok_v, sr_v, sem):
        return pltpu.async_copy(skill_t.at[tok_v], sr_v, sem)

    def wait_gather(tok_v, sr_v, sem):
        pltpu.make_async_copy(skill_t.at[tok_v], sr_v, sem).wait()

    def wait_tok(c, tok_v, sem):
        pltpu.make_async_copy(
            tok_hbm.at[pl.ds(sbase + c * SK_CHUNK, SK_CHUNK)], tok_v,
            sem).wait()

    def reduce_chunk(c, sr_v):
        for s in range(SPC):
            r0 = s * SKILL_LEN
            row = c * SPC + s
            p0, p1 = [], []
            for g in range(SKILL_LEN // GRP):
                b0 = r0 + g * GRP
                a0, a1 = plsc.unpack(sr_v[b0, :],
                                     format=plsc.PackFormat.INTERLEAVED,
                                     preferred_element_type=jnp.float32)
                for j in range(1, GRP):
                    u0, u1 = plsc.unpack(sr_v[b0 + j, :],
                                         format=plsc.PackFormat.INTERLEAVED,
                                         preferred_element_type=jnp.float32)
                    a0 = a0 + u0
                    a1 = a1 + u1
                p0.append(a0)
                p1.append(a1)
            t0 = (p0[0] + p0[1]) + (p0[2] + p0[3]) + p0[4]
            t1 = (p1[0] + p1[1]) + (p1[2] + p1[3]) + p1[4]
            out_v[row, :] = plsc.pack(t0, t1,
                                      format=plsc.PackFormat.INTERLEAVED,
                                      preferred_element_type=jnp.bfloat16)

    load_tok(0, tok0, lsem0).wait()
    fire_gather(tok0, sr0, gsem0)
    load_tok(1, tok1, lsem1).wait()
    fire_gather(tok1, sr1, gsem1)

    @pl.loop(0, NCHUNK - 2, step=2)
    def _(c):
        wait_gather(tok0, sr0, gsem0)
        load_tok(c + 2, tok0, lsem0)
        reduce_chunk(c, sr0)
        wait_tok(c + 2, tok0, lsem0)
        fire_gather(tok0, sr0, gsem0)
        wait_gather(tok1, sr1, gsem1)
        load_tok(c + 3, tok1, lsem1)
        reduce_chunk(c + 1, sr1)
        wait_tok(c + 3, tok1, lsem1)
        fire_gather(tok1, sr1, gsem1)

    wait_gather(tok0, sr0, gsem0)
    reduce_chunk(NCHUNK - 2, sr0)
    wait_gather(tok1, sr1, gsem1)
    reduce_chunk(NCHUNK - 1, sr1)

    pltpu.sync_copy(out_v, feat_hbm.at[rows, pl.ds(160, 32)])
    w_loc.wait()
    w_cat.wait()
    w_lev.wait()


@jax.jit
def _sc_gather(job_id, category, location, level, tok_flat,
               job_t, cat_t, loc_t, lev_t, skill_t):
    bf16 = jnp.bfloat16
    i32 = jnp.int32
    out_type = jax.ShapeDtypeStruct((B, FEAT), bf16)
    scratch = [
        pltpu.VMEM((BPW,), i32),
        pltpu.VMEM((BPW,), i32),
        pltpu.VMEM((BPW, 32), bf16),
        pltpu.VMEM((BPW, 32), bf16),
        pltpu.VMEM((BPW, 64), bf16),
        pltpu.VMEM((SK_CHUNK,), i32),
        pltpu.VMEM((SK_CHUNK,), i32),
        pltpu.VMEM((SK_CHUNK, 32), bf16),
        pltpu.VMEM((SK_CHUNK, 32), bf16),
        pltpu.VMEM((BPW, 32), bf16),
        pltpu.SemaphoreType.DMA,
        pltpu.SemaphoreType.DMA,
        pltpu.SemaphoreType.DMA,
        pltpu.SemaphoreType.DMA,
        pltpu.SemaphoreType.DMA,
    ]
    mesh = plsc.VectorSubcoreMesh(core_axis_name="c", subcore_axis_name="s")
    k = pl.kernel(_sc_gather_body, out_type=out_type, mesh=mesh,
                  scratch_types=scratch,
                  compiler_params=pltpu.CompilerParams(
                      use_tc_tiling_on_sc=False,
                      needs_layout_passes=False))
    return k(job_id, category, location, level, tok_flat,
             job_t, cat_t, loc_t, lev_t, skill_t)


BB = 512


def _mlp_body(feat, W1, b1, W2, b2, W3, b3, out):
    h = jnp.maximum(jnp.dot(feat[...], W1[...],
                            preferred_element_type=jnp.float32) + b1[...], 0.0)
    h = jnp.maximum(jnp.dot(h.astype(jnp.bfloat16), W2[...],
                            preferred_element_type=jnp.float32) + b2[...], 0.0)
    out[...] = jnp.dot(h.astype(jnp.bfloat16), W3[...],
                       preferred_element_type=jnp.float32) + b3[...]


@jax.jit
def _tc_mlp(feat, W1, b1, W2, b2, W3, b3):
    nb = B // BB
    full = lambda a: pl.BlockSpec(a.shape, lambda i: tuple(0 for _ in a.shape))
    return pl.pallas_call(
        _mlp_body,
        grid=(nb,),
        in_specs=[pl.BlockSpec((BB, FEAT), lambda i: (i, 0)),
                  full(W1), full(b1), full(W2), full(b2), full(W3), full(b3)],
        out_specs=pl.BlockSpec((BB, 64), lambda i: (i, 0)),
        out_shape=jax.ShapeDtypeStruct((B, 64), jnp.float32),
    )(feat, W1, b1, W2, b2, W3, b3)


def kernel(job_id, category, location, level, skill_tokens,
           job_table, category_table, location_table, level_table, skill_table,
           W1, b1, W2, b2, W3, b3):
    bf16 = jnp.bfloat16
    feat = _sc_gather(job_id, category, location, level,
                      skill_tokens.reshape(-1),
                      job_table.astype(bf16), category_table.astype(bf16),
                      location_table.astype(bf16), level_table.astype(bf16),
                      skill_table.astype(bf16))
    scale = jnp.concatenate([jnp.ones((160,), jnp.float32),
                             jnp.full((32,), 1.0 / SKILL_LEN, jnp.float32)])
    W1s = (W1 * scale[:, None]).astype(bf16)
    return _tc_mlp(feat, W1s, b1, W2.astype(bf16), b2, W3.astype(bf16), b3)


# speedup vs baseline: 1.5303x; 1.2675x over previous
import functools

import jax
import jax.numpy as jnp
from jax import lax
from jax.experimental import pallas as pl
from jax.experimental.pallas import tpu as pltpu
from jax.experimental.pallas import tpu_sc as plsc

B = 16384
SKILL_LEN = 50
NC, NS = 2, 16
NW = NC * NS
BPW = B // NW
SPC = 8
SK_CHUNK = SPC * SKILL_LEN
NCHUNK = BPW // SPC
GRP = 10

_SC_PARAMS = dict(
    compiler_params=pltpu.CompilerParams(use_tc_tiling_on_sc=False,
                                         needs_layout_passes=False))


def _sc_features_body(job_hbm, cat_hbm, loc_hbm, lev_hbm,
                      job_t, cat_t, loc_t, lev_t,
                      feat_hbm,
                      idxa, idxb, r32a, r32b, r64,
                      gsem0, gsem1, wsem):
    sid = lax.axis_index("s")
    wid = sid * NC + lax.axis_index("c")
    rows = pl.ds(wid * BPW, BPW)

    pltpu.sync_copy(job_hbm.at[rows], idxa)
    g_job = pltpu.async_copy(job_t.at[idxa], r32a, gsem0)
    pltpu.sync_copy(loc_hbm.at[rows], idxb)
    g_loc = pltpu.async_copy(loc_t.at[idxb], r32b, gsem1)
    g_job.wait()
    w_job = pltpu.async_copy(r32a, feat_hbm.at[rows, pl.ds(0, 32)], wsem)
    g_loc.wait()
    w_loc = pltpu.async_copy(r32b, feat_hbm.at[rows, pl.ds(96, 32)], wsem)
    pltpu.sync_copy(cat_hbm.at[rows], idxa)
    g_cat = pltpu.async_copy(cat_t.at[idxa], r64, gsem0)
    pltpu.sync_copy(lev_hbm.at[rows], idxb)
    w_job.wait()
    g_lev = pltpu.async_copy(lev_t.at[idxb], r32a, gsem1)
    g_cat.wait()
    w_cat = pltpu.async_copy(r64, feat_hbm.at[rows, pl.ds(32, 64)], wsem)
    g_lev.wait()
    w_lev = pltpu.async_copy(r32a, feat_hbm.at[rows, pl.ds(128, 32)], wsem)
    w_loc.wait()
    w_cat.wait()
    w_lev.wait()


@jax.jit
def _sc_features(job_id, category, location, level,
                 job_t, cat_t, loc_t, lev_t):
    bf16 = jnp.bfloat16
    i32 = jnp.int32
    scratch = [
        pltpu.VMEM((BPW,), i32),
        pltpu.VMEM((BPW,), i32),
        pltpu.VMEM((BPW, 32), bf16),
        pltpu.VMEM((BPW, 32), bf16),
        pltpu.VMEM((BPW, 64), bf16),
        pltpu.SemaphoreType.DMA,
        pltpu.SemaphoreType.DMA,
        pltpu.SemaphoreType.DMA,
    ]
    mesh = plsc.VectorSubcoreMesh(core_axis_name="c", subcore_axis_name="s")
    k = pl.kernel(_sc_features_body,
                  out_type=jax.ShapeDtypeStruct((B, 160), bf16),
                  mesh=mesh, scratch_types=scratch, **_SC_PARAMS)
    return k(job_id, category, location, level, job_t, cat_t, loc_t, lev_t)


def _sc_skill_body(tok_hbm, skill_t, out_hbm,
                   tok0, tok1, sr0, sr1, out_v,
                   gsem0, gsem1, lsem0, lsem1):
    sid = lax.axis_index("s")
    wid = sid * NC + lax.axis_index("c")
    sbase = wid * BPW * SKILL_LEN

    def load_tok(c, tok_v, sem):
        return pltpu.async_copy(
            tok_hbm.at[pl.ds(sbase + c * SK_CHUNK, SK_CHUNK)], tok_v, sem)

    def fire_gather(tok_v, sr_v, sem):
        return pltpu.async_copy(skill_t.at[tok_v], sr_v, sem)

    def wait_gather(tok_v, sr_v, sem):
        pltpu.make_async_copy(skill_t.at[tok_v], sr_v, sem).wait()

    def wait_tok(c, tok_v, sem):
        pltpu.make_async_copy(
            tok_hbm.at[pl.ds(sbase + c * SK_CHUNK, SK_CHUNK)], tok_v,
            sem).wait()

    def reduce_chunk(c, sr_v):
        for s in range(SPC):
            r0 = s * SKILL_LEN
            row = c * SPC + s
            p0, p1 = [], []
            for g in range(SKILL_LEN // GRP):
                b0 = r0 + g * GRP
                a0, a1 = plsc.unpack(sr_v[b0, :],
                                     format=plsc.PackFormat.INTERLEAVED,
                                     preferred_element_type=jnp.float32)
                for j in range(1, GRP):
                    u0, u1 = plsc.unpack(sr_v[b0 + j, :],
                                         format=plsc.PackFormat.INTERLEAVED,
                                         preferred_element_type=jnp.float32)
                    a0 = a0 + u0
                    a1 = a1 + u1
                p0.append(a0)
                p1.append(a1)
            t0 = (p0[0] + p0[1]) + (p0[2] + p0[3]) + p0[4]
            t1 = (p1[0] + p1[1]) + (p1[2] + p1[3]) + p1[4]
            out_v[row, :] = plsc.pack(t0, t1,
                                      format=plsc.PackFormat.INTERLEAVED,
                                      preferred_element_type=jnp.bfloat16)

    load_tok(0, tok0, lsem0).wait()
    fire_gather(tok0, sr0, gsem0)
    load_tok(1, tok1, lsem1).wait()
    fire_gather(tok1, sr1, gsem1)

    @pl.loop(0, NCHUNK - 2, step=2)
    def _(c):
        wait_gather(tok0, sr0, gsem0)
        load_tok(c + 2, tok0, lsem0)
        reduce_chunk(c, sr0)
        wait_tok(c + 2, tok0, lsem0)
        fire_gather(tok0, sr0, gsem0)
        wait_gather(tok1, sr1, gsem1)
        load_tok(c + 3, tok1, lsem1)
        reduce_chunk(c + 1, sr1)
        wait_tok(c + 3, tok1, lsem1)
        fire_gather(tok1, sr1, gsem1)

    wait_gather(tok0, sr0, gsem0)
    reduce_chunk(NCHUNK - 2, sr0)
    wait_gather(tok1, sr1, gsem1)
    reduce_chunk(NCHUNK - 1, sr1)

    pltpu.sync_copy(out_v, out_hbm.at[pl.ds(wid * BPW, BPW), :])


@jax.jit
def _sc_skill(tok_flat, skill_t):
    bf16 = jnp.bfloat16
    i32 = jnp.int32
    scratch = [
        pltpu.VMEM((SK_CHUNK,), i32),
        pltpu.VMEM((SK_CHUNK,), i32),
        pltpu.VMEM((SK_CHUNK, 32), bf16),
        pltpu.VMEM((SK_CHUNK, 32), bf16),
        pltpu.VMEM((BPW, 32), bf16),
        pltpu.SemaphoreType.DMA,
        pltpu.SemaphoreType.DMA,
        pltpu.SemaphoreType.DMA,
        pltpu.SemaphoreType.DMA,
    ]
    mesh = plsc.VectorSubcoreMesh(core_axis_name="c", subcore_axis_name="s")
    k = pl.kernel(_sc_skill_body,
                  out_type=jax.ShapeDtypeStruct((B, 32), bf16),
                  mesh=mesh, scratch_types=scratch, **_SC_PARAMS)
    return k(tok_flat, skill_t)


BB = 1024


def _mlp_body(fa, sk, W1a, W1b, b1, W2, b2, W3, b3, out):
    h = jnp.dot(fa[...], W1a[...], preferred_element_type=jnp.float32)
    h = h + jnp.dot(sk[...], W1b[...], preferred_element_type=jnp.float32)
    h = jnp.maximum(h + b1[...], 0.0)
    h = jnp.maximum(jnp.dot(h.astype(jnp.bfloat16), W2[...],
                            preferred_element_type=jnp.float32) + b2[...], 0.0)
    out[...] = jnp.dot(h.astype(jnp.bfloat16), W3[...],
                       preferred_element_type=jnp.float32) + b3[...]


@jax.jit
def _tc_mlp(fa, sk, W1a, W1b, b1, W2, b2, W3, b3):
    nb = B // BB
    full = lambda a: pl.BlockSpec(a.shape, lambda i: tuple(0 for _ in a.shape))
    return pl.pallas_call(
        _mlp_body,
        grid=(nb,),
        in_specs=[pl.BlockSpec((BB, 160), lambda i: (i, 0)),
                  pl.BlockSpec((BB, 32), lambda i: (i, 0)),
                  full(W1a), full(W1b), full(b1),
                  full(W2), full(b2), full(W3), full(b3)],
        out_specs=pl.BlockSpec((BB, 64), lambda i: (i, 0)),
        out_shape=jax.ShapeDtypeStruct((B, 64), jnp.float32),
    )(fa, sk, W1a, W1b, b1, W2, b2, W3, b3)


def kernel(job_id, category, location, level, skill_tokens,
           job_table, category_table, location_table, level_table, skill_table,
           W1, b1, W2, b2, W3, b3):
    bf16 = jnp.bfloat16
    fa = _sc_features(job_id, category, location, level,
                      job_table.astype(bf16), category_table.astype(bf16),
                      location_table.astype(bf16), level_table.astype(bf16))
    sk = _sc_skill(skill_tokens.reshape(-1), skill_table.astype(bf16))
    W1a = W1[:160].astype(bf16)
    W1b = (W1[160:] * (1.0 / SKILL_LEN)).astype(bf16)
    return _tc_mlp(fa, sk, W1a, W1b, b1,
                   W2.astype(bf16), b2, W3.astype(bf16), b3)


# speedup vs baseline: 2.0354x; 1.3301x over previous
import functools

import jax
import jax.numpy as jnp
from jax import lax
from jax.experimental import pallas as pl
from jax.experimental.pallas import tpu as pltpu
from jax.experimental.pallas import tpu_sc as plsc

B = 16384
SKILL_LEN = 50
NC, NS = 2, 16
NW = NC * NS
BPW = B // NW
SPC = 8
SK_CHUNK = SPC * SKILL_LEN
NCHUNK = BPW // SPC
GRP = 10

_SC_PARAMS = dict(
    compiler_params=pltpu.CompilerParams(use_tc_tiling_on_sc=False,
                                         needs_layout_passes=False))


def _sc_features_body(job_hbm, cat_hbm, loc_hbm,
                      job_t, cat_t, loc_t,
                      feat_hbm,
                      idxa, idxb, r32a, r32b, r64,
                      gsem0, gsem1, wsem):
    sid = lax.axis_index("s")
    wid = sid * NC + lax.axis_index("c")
    rows = pl.ds(wid * BPW, BPW)

    pltpu.sync_copy(job_hbm.at[rows], idxa)
    g_job = pltpu.async_copy(job_t.at[idxa], r32a, gsem0)
    pltpu.sync_copy(loc_hbm.at[rows], idxb)
    g_loc = pltpu.async_copy(loc_t.at[idxb], r32b, gsem1)
    pltpu.sync_copy(cat_hbm.at[rows], idxa)
    g_cat = pltpu.async_copy(cat_t.at[idxa], r64, gsem0)
    g_job.wait()
    w_job = pltpu.async_copy(r32a, feat_hbm.at[rows, pl.ds(0, 32)], wsem)
    g_loc.wait()
    w_loc = pltpu.async_copy(r32b, feat_hbm.at[rows, pl.ds(96, 32)], wsem)
    g_cat.wait()
    w_cat = pltpu.async_copy(r64, feat_hbm.at[rows, pl.ds(32, 64)], wsem)
    w_job.wait()
    w_loc.wait()
    w_cat.wait()


@jax.jit
def _sc_features(job_id, category, location, job_t, cat_t, loc_t):
    bf16 = jnp.bfloat16
    i32 = jnp.int32
    scratch = [
        pltpu.VMEM((BPW,), i32),
        pltpu.VMEM((BPW,), i32),
        pltpu.VMEM((BPW, 32), bf16),
        pltpu.VMEM((BPW, 32), bf16),
        pltpu.VMEM((BPW, 64), bf16),
        pltpu.SemaphoreType.DMA,
        pltpu.SemaphoreType.DMA,
        pltpu.SemaphoreType.DMA,
    ]
    mesh = plsc.VectorSubcoreMesh(core_axis_name="c", subcore_axis_name="s")
    k = pl.kernel(_sc_features_body,
                  out_type=jax.ShapeDtypeStruct((B, 128), bf16),
                  mesh=mesh, scratch_types=scratch, **_SC_PARAMS)
    return k(job_id, category, location, job_t, cat_t, loc_t)


def _sc_skill_body(tok_hbm, skill_t, out_hbm,
                   tok0, tok1, sr0, sr1, out_v,
                   gsem0, gsem1, lsem0, lsem1):
    sid = lax.axis_index("s")
    wid = sid * NC + lax.axis_index("c")
    sbase = wid * BPW * SKILL_LEN

    def load_tok(c, tok_v, sem):
        return pltpu.async_copy(
            tok_hbm.at[pl.ds(sbase + c * SK_CHUNK, SK_CHUNK)], tok_v, sem)

    def fire_gather(tok_v, sr_v, sem):
        return pltpu.async_copy(skill_t.at[tok_v], sr_v, sem)

    def wait_gather(tok_v, sr_v, sem):
        pltpu.make_async_copy(skill_t.at[tok_v], sr_v, sem).wait()

    def wait_tok(c, tok_v, sem):
        pltpu.make_async_copy(
            tok_hbm.at[pl.ds(sbase + c * SK_CHUNK, SK_CHUNK)], tok_v,
            sem).wait()

    def reduce_chunk(c, sr_v):
        for s in range(SPC):
            r0 = s * SKILL_LEN
            row = c * SPC + s
            p0, p1 = [], []
            for g in range(SKILL_LEN // GRP):
                b0 = r0 + g * GRP
                a0, a1 = plsc.unpack(sr_v[b0, :],
                                     format=plsc.PackFormat.INTERLEAVED,
                                     preferred_element_type=jnp.float32)
                for j in range(1, GRP):
                    u0, u1 = plsc.unpack(sr_v[b0 + j, :],
                                         format=plsc.PackFormat.INTERLEAVED,
                                         preferred_element_type=jnp.float32)
                    a0 = a0 + u0
                    a1 = a1 + u1
                p0.append(a0)
                p1.append(a1)
            t0 = (p0[0] + p0[1]) + (p0[2] + p0[3]) + p0[4]
            t1 = (p1[0] + p1[1]) + (p1[2] + p1[3]) + p1[4]
            out_v[row, :] = plsc.pack(t0, t1,
                                      format=plsc.PackFormat.INTERLEAVED,
                                      preferred_element_type=jnp.bfloat16)

    load_tok(0, tok0, lsem0).wait()
    fire_gather(tok0, sr0, gsem0)
    load_tok(1, tok1, lsem1).wait()
    fire_gather(tok1, sr1, gsem1)

    @pl.loop(0, NCHUNK - 2, step=2)
    def _(c):
        wait_gather(tok0, sr0, gsem0)
        load_tok(c + 2, tok0, lsem0)
        reduce_chunk(c, sr0)
        wait_tok(c + 2, tok0, lsem0)
        fire_gather(tok0, sr0, gsem0)
        wait_gather(tok1, sr1, gsem1)
        load_tok(c + 3, tok1, lsem1)
        reduce_chunk(c + 1, sr1)
        wait_tok(c + 3, tok1, lsem1)
        fire_gather(tok1, sr1, gsem1)

    wait_gather(tok0, sr0, gsem0)
    reduce_chunk(NCHUNK - 2, sr0)
    wait_gather(tok1, sr1, gsem1)
    reduce_chunk(NCHUNK - 1, sr1)

    pltpu.sync_copy(out_v, out_hbm.at[pl.ds(wid * BPW, BPW), :])


@jax.jit
def _sc_skill(tok_flat, skill_t):
    bf16 = jnp.bfloat16
    i32 = jnp.int32
    scratch = [
        pltpu.VMEM((SK_CHUNK,), i32),
        pltpu.VMEM((SK_CHUNK,), i32),
        pltpu.VMEM((SK_CHUNK, 32), bf16),
        pltpu.VMEM((SK_CHUNK, 32), bf16),
        pltpu.VMEM((BPW, 32), bf16),
        pltpu.SemaphoreType.DMA,
        pltpu.SemaphoreType.DMA,
        pltpu.SemaphoreType.DMA,
        pltpu.SemaphoreType.DMA,
    ]
    mesh = plsc.VectorSubcoreMesh(core_axis_name="c", subcore_axis_name="s")
    k = pl.kernel(_sc_skill_body,
                  out_type=jax.ShapeDtypeStruct((B, 32), bf16),
                  mesh=mesh, scratch_types=scratch, **_SC_PARAMS)
    return k(tok_flat, skill_t)


BB = 1024


def _mlp_body(fa, sk, lev, W1a, W1b, Wlev, b1, W2, b2, W3, b3, out):
    h = jnp.dot(fa[...], W1a[...], preferred_element_type=jnp.float32)
    h = h + jnp.dot(sk[...], W1b[...], preferred_element_type=jnp.float32)
    ids = lev[...]
    onehot = (jax.lax.broadcasted_iota(jnp.int32, (BB, 16), 1)
              == ids[:, None]).astype(jnp.bfloat16)
    h = h + jnp.dot(onehot, Wlev[...], preferred_element_type=jnp.float32)
    h = jnp.maximum(h + b1[...], 0.0)
    h = jnp.maximum(jnp.dot(h.astype(jnp.bfloat16), W2[...],
                            preferred_element_type=jnp.float32) + b2[...], 0.0)
    out[...] = jnp.dot(h.astype(jnp.bfloat16), W3[...],
                       preferred_element_type=jnp.float32) + b3[...]


@jax.jit
def _tc_mlp(fa, sk, lev, W1a, W1b, Wlev, b1, W2, b2, W3, b3):
    nb = B // BB
    full = lambda a: pl.BlockSpec(a.shape, lambda i: tuple(0 for _ in a.shape))
    return pl.pallas_call(
        _mlp_body,
        grid=(nb,),
        in_specs=[pl.BlockSpec((BB, 128), lambda i: (i, 0)),
                  pl.BlockSpec((BB, 32), lambda i: (i, 0)),
                  pl.BlockSpec((BB,), lambda i: (i,)),
                  full(W1a), full(W1b), full(Wlev), full(b1),
                  full(W2), full(b2), full(W3), full(b3)],
        out_specs=pl.BlockSpec((BB, 64), lambda i: (i, 0)),
        out_shape=jax.ShapeDtypeStruct((B, 64), jnp.float32),
    )(fa, sk, lev, W1a, W1b, Wlev, b1, W2, b2, W3, b3)


def kernel(job_id, category, location, level, skill_tokens,
           job_table, category_table, location_table, level_table, skill_table,
           W1, b1, W2, b2, W3, b3):
    bf16 = jnp.bfloat16
    fa = _sc_features(job_id, category, location,
                      job_table.astype(bf16), category_table.astype(bf16),
                      location_table.astype(bf16))
    tok_flat, _ = jax.lax.optimization_barrier(
        (skill_tokens.reshape(-1), fa[0, 0]))
    sk = _sc_skill(tok_flat, skill_table.astype(bf16))
    W1a = W1[:128].astype(bf16)
    W1b = (W1[160:] * (1.0 / SKILL_LEN)).astype(bf16)
    Wlev = jnp.zeros((16, 256), jnp.float32).at[:11].set(
        level_table @ W1[128:160]).astype(bf16)
    return _tc_mlp(fa, sk, level, W1a, W1b, Wlev, b1,
                   W2.astype(bf16), b2, W3.astype(bf16), b3)


# speedup vs baseline: 2.0698x; 1.0169x over previous
import functools

import jax
import jax.numpy as jnp
from jax import lax
from jax.experimental import pallas as pl
from jax.experimental.pallas import tpu as pltpu
from jax.experimental.pallas import tpu_sc as plsc

B = 16384
SKILL_LEN = 50
NC, NS = 2, 16
NW = NC * NS
BPW = B // NW
SPC = 8
SK_CHUNK = SPC * SKILL_LEN
NCHUNK = BPW // SPC
GRP = 10

_SC_PARAMS = dict(
    compiler_params=pltpu.CompilerParams(use_tc_tiling_on_sc=False,
                                         needs_layout_passes=False))


def _sc_features_body(job_hbm, cat_hbm, loc_hbm,
                      job_t, cat_t, loc_t,
                      feat_hbm,
                      idxa, idxb, r32a, r32b, r64,
                      gsem0, gsem1, wsem):
    sid = lax.axis_index("s")
    wid = sid * NC + lax.axis_index("c")
    rows = pl.ds(wid * BPW, BPW)

    pltpu.sync_copy(job_hbm.at[rows], idxa)
    g_job = pltpu.async_copy(job_t.at[idxa], r32a, gsem0)
    pltpu.sync_copy(loc_hbm.at[rows], idxb)
    g_loc = pltpu.async_copy(loc_t.at[idxb], r32b, gsem1)
    pltpu.sync_copy(cat_hbm.at[rows], idxa)
    g_cat = pltpu.async_copy(cat_t.at[idxa], r64, gsem0)
    g_job.wait()
    w_job = pltpu.async_copy(r32a, feat_hbm.at[rows, pl.ds(0, 32)], wsem)
    g_loc.wait()
    w_loc = pltpu.async_copy(r32b, feat_hbm.at[rows, pl.ds(96, 32)], wsem)
    g_cat.wait()
    w_cat = pltpu.async_copy(r64, feat_hbm.at[rows, pl.ds(32, 64)], wsem)
    w_job.wait()
    w_loc.wait()
    w_cat.wait()


@jax.jit
def _sc_features(job_id, category, location, job_t, cat_t, loc_t):
    bf16 = jnp.bfloat16
    i32 = jnp.int32
    scratch = [
        pltpu.VMEM((BPW,), i32),
        pltpu.VMEM((BPW,), i32),
        pltpu.VMEM((BPW, 32), bf16),
        pltpu.VMEM((BPW, 32), bf16),
        pltpu.VMEM((BPW, 64), bf16),
        pltpu.SemaphoreType.DMA,
        pltpu.SemaphoreType.DMA,
        pltpu.SemaphoreType.DMA,
    ]
    mesh = plsc.VectorSubcoreMesh(core_axis_name="c", subcore_axis_name="s")
    k = pl.kernel(_sc_features_body,
                  out_type=jax.ShapeDtypeStruct((B, 128), bf16),
                  mesh=mesh, scratch_types=scratch, **_SC_PARAMS)
    return k(job_id, category, location, job_t, cat_t, loc_t)


def _sc_skill_body(tok_hbm, skill_t, out_hbm,
                   tok0, tok1, sr0, sr1, out_v,
                   gsem0, gsem1, lsem0, lsem1):
    sid = lax.axis_index("s")
    wid = sid * NC + lax.axis_index("c")
    sbase = wid * BPW * SKILL_LEN

    def load_tok(c, tok_v, sem):
        return pltpu.async_copy(
            tok_hbm.at[pl.ds(sbase + c * SK_CHUNK, SK_CHUNK)], tok_v, sem)

    def fire_gather(tok_v, sr_v, sem):
        return pltpu.async_copy(skill_t.at[tok_v], sr_v, sem)

    def wait_gather(tok_v, sr_v, sem):
        pltpu.make_async_copy(skill_t.at[tok_v], sr_v, sem).wait()

    def wait_tok(c, tok_v, sem):
        pltpu.make_async_copy(
            tok_hbm.at[pl.ds(sbase + c * SK_CHUNK, SK_CHUNK)], tok_v,
            sem).wait()

    def reduce_chunk(c, sr_v):
        for s in range(SPC):
            r0 = s * SKILL_LEN
            row = c * SPC + s
            p0, p1 = [], []
            for g in range(SKILL_LEN // GRP):
                b0 = r0 + g * GRP
                a0, a1 = plsc.unpack(sr_v[b0, :],
                                     format=plsc.PackFormat.INTERLEAVED,
                                     preferred_element_type=jnp.float32)
                for j in range(1, GRP):
                    u0, u1 = plsc.unpack(sr_v[b0 + j, :],
                                         format=plsc.PackFormat.INTERLEAVED,
                                         preferred_element_type=jnp.float32)
                    a0 = a0 + u0
                    a1 = a1 + u1
                p0.append(a0)
                p1.append(a1)
            t0 = (p0[0] + p0[1]) + (p0[2] + p0[3]) + p0[4]
            t1 = (p1[0] + p1[1]) + (p1[2] + p1[3]) + p1[4]
            out_v[row, :] = plsc.pack(t0, t1,
                                      format=plsc.PackFormat.INTERLEAVED,
                                      preferred_element_type=jnp.bfloat16)

    load_tok(0, tok0, lsem0).wait()
    fire_gather(tok0, sr0, gsem0)
    load_tok(1, tok1, lsem1).wait()
    fire_gather(tok1, sr1, gsem1)

    @pl.loop(0, NCHUNK - 2, step=2)
    def _(c):
        wait_gather(tok0, sr0, gsem0)
        load_tok(c + 2, tok0, lsem0)
        reduce_chunk(c, sr0)
        wait_tok(c + 2, tok0, lsem0)
        fire_gather(tok0, sr0, gsem0)
        wait_gather(tok1, sr1, gsem1)
        load_tok(c + 3, tok1, lsem1)
        reduce_chunk(c + 1, sr1)
        wait_tok(c + 3, tok1, lsem1)
        fire_gather(tok1, sr1, gsem1)

    wait_gather(tok0, sr0, gsem0)
    reduce_chunk(NCHUNK - 2, sr0)
    wait_gather(tok1, sr1, gsem1)
    reduce_chunk(NCHUNK - 1, sr1)

    pltpu.sync_copy(out_v, out_hbm.at[pl.ds(wid * BPW, BPW), :])


@jax.jit
def _sc_skill(tok_flat, skill_t):
    bf16 = jnp.bfloat16
    i32 = jnp.int32
    scratch = [
        pltpu.VMEM((SK_CHUNK,), i32),
        pltpu.VMEM((SK_CHUNK,), i32),
        pltpu.VMEM((SK_CHUNK, 32), bf16),
        pltpu.VMEM((SK_CHUNK, 32), bf16),
        pltpu.VMEM((BPW, 32), bf16),
        pltpu.SemaphoreType.DMA,
        pltpu.SemaphoreType.DMA,
        pltpu.SemaphoreType.DMA,
        pltpu.SemaphoreType.DMA,
    ]
    mesh = plsc.VectorSubcoreMesh(core_axis_name="c", subcore_axis_name="s")
    k = pl.kernel(_sc_skill_body,
                  out_type=jax.ShapeDtypeStruct((B, 32), bf16),
                  mesh=mesh, scratch_types=scratch, **_SC_PARAMS)
    return k(tok_flat, skill_t)


BB = 2048


def _mlp_body(fa, sk, lev, W1a, W1b, Wlev, b1, W2, b2, W3, b3, out):
    h = jnp.dot(fa[...], W1a[...], preferred_element_type=jnp.float32)
    h = h + jnp.dot(sk[...], W1b[...], preferred_element_type=jnp.float32)
    ids = lev[...]
    onehot = (jax.lax.broadcasted_iota(jnp.int32, (BB, 16), 1)
              == ids[:, None]).astype(jnp.bfloat16)
    h = h + jnp.dot(onehot, Wlev[...], preferred_element_type=jnp.float32)
    h = jnp.maximum(h + b1[...], 0.0)
    h = jnp.maximum(jnp.dot(h.astype(jnp.bfloat16), W2[...],
                            preferred_element_type=jnp.float32) + b2[...], 0.0)
    out[...] = jnp.dot(h.astype(jnp.bfloat16), W3[...],
                       preferred_element_type=jnp.float32) + b3[...]


@jax.jit
def _tc_mlp(fa, sk, lev, W1a, W1b, Wlev, b1, W2, b2, W3, b3):
    nb = B // BB
    full = lambda a: pl.BlockSpec(a.shape, lambda i: tuple(0 for _ in a.shape))
    return pl.pallas_call(
        _mlp_body,
        grid=(nb,),
        in_specs=[pl.BlockSpec((BB, 128), lambda i: (i, 0)),
                  pl.BlockSpec((BB, 32), lambda i: (i, 0)),
                  pl.BlockSpec((BB,), lambda i: (i,)),
                  full(W1a), full(W1b), full(Wlev), full(b1),
                  full(W2), full(b2), full(W3), full(b3)],
        out_specs=pl.BlockSpec((BB, 64), lambda i: (i, 0)),
        out_shape=jax.ShapeDtypeStruct((B, 64), jnp.float32),
    )(fa, sk, lev, W1a, W1b, Wlev, b1, W2, b2, W3, b3)


def kernel(job_id, category, location, level, skill_tokens,
           job_table, category_table, location_table, level_table, skill_table,
           W1, b1, W2, b2, W3, b3):
    bf16 = jnp.bfloat16
    fa = _sc_features(job_id, category, location,
                      job_table.astype(bf16), category_table.astype(bf16),
                      location_table.astype(bf16))
    sk = _sc_skill(skill_tokens.reshape(-1), skill_table.astype(bf16))
    W1a = W1[:128].astype(bf16)
    W1b = (W1[160:] * (1.0 / SKILL_LEN)).astype(bf16)
    Wlev = jnp.zeros((16, 256), jnp.float32).at[:11].set(
        level_table @ W1[128:160]).astype(bf16)
    return _tc_mlp(fa, sk, level, W1a, W1b, Wlev, b1,
                   W2.astype(bf16), b2, W3.astype(bf16), b3)


# speedup vs baseline: 2.1639x; 1.0455x over previous
import functools

import jax
import jax.numpy as jnp
from jax import lax
from jax.experimental import pallas as pl
from jax.experimental.pallas import tpu as pltpu
from jax.experimental.pallas import tpu_sc as plsc

B = 16384
SKILL_LEN = 50
NC, NS = 2, 16
NW = NC * NS
BPW = B // NW
SPC = 8
SK_CHUNK = SPC * SKILL_LEN
NCHUNK = BPW // SPC
GRP = 10

_SC_PARAMS = dict(
    compiler_params=pltpu.CompilerParams(use_tc_tiling_on_sc=False,
                                         needs_layout_passes=False))


def _sc_features_body(job_hbm, cat_hbm, loc_hbm,
                      job_t, cat_t, loc_t,
                      feat_hbm,
                      idxa, idxb, r32a, r32b, r64,
                      gsem0, gsem1, wsem):
    sid = lax.axis_index("s")
    wid = sid * NC + lax.axis_index("c")
    rows = pl.ds(wid * BPW, BPW)

    pltpu.sync_copy(job_hbm.at[rows], idxa)
    g_job = pltpu.async_copy(job_t.at[idxa], r32a, gsem0)
    pltpu.sync_copy(loc_hbm.at[rows], idxb)
    g_loc = pltpu.async_copy(loc_t.at[idxb], r32b, gsem1)
    pltpu.sync_copy(cat_hbm.at[rows], idxa)
    g_cat = pltpu.async_copy(cat_t.at[idxa], r64, gsem0)
    g_job.wait()
    w_job = pltpu.async_copy(r32a, feat_hbm.at[rows, pl.ds(0, 32)], wsem)
    g_loc.wait()
    w_loc = pltpu.async_copy(r32b, feat_hbm.at[rows, pl.ds(96, 32)], wsem)
    g_cat.wait()
    w_cat = pltpu.async_copy(r64, feat_hbm.at[rows, pl.ds(32, 64)], wsem)
    w_job.wait()
    w_loc.wait()
    w_cat.wait()


@jax.jit
def _sc_features(job_id, category, location, job_t, cat_t, loc_t):
    bf16 = jnp.bfloat16
    i32 = jnp.int32
    scratch = [
        pltpu.VMEM((BPW,), i32),
        pltpu.VMEM((BPW,), i32),
        pltpu.VMEM((BPW, 32), bf16),
        pltpu.VMEM((BPW, 32), bf16),
        pltpu.VMEM((BPW, 64), bf16),
        pltpu.SemaphoreType.DMA,
        pltpu.SemaphoreType.DMA,
        pltpu.SemaphoreType.DMA,
    ]
    mesh = plsc.VectorSubcoreMesh(core_axis_name="c", subcore_axis_name="s")
    k = pl.kernel(_sc_features_body,
                  out_type=jax.ShapeDtypeStruct((B, 128), bf16),
                  mesh=mesh, scratch_types=scratch, **_SC_PARAMS)
    return k(job_id, category, location, job_t, cat_t, loc_t)


def _sc_skill_body(tok_hbm, skill_t_hbm, out_hbm,
                   tok0, tok1, sr0, sr1, out_v, skill_t,
                   gsem0, gsem1, lsem0, lsem1):
    sid = lax.axis_index("s")
    wid = sid * NC + lax.axis_index("c")
    vrows = 10000 // NS
    pltpu.sync_copy(skill_t_hbm.at[pl.ds(sid * vrows, vrows), :],
                    skill_t.at[pl.ds(sid * vrows, vrows), :])
    plsc.subcore_barrier()
    sbase = wid * BPW * SKILL_LEN

    def load_tok(c, tok_v, sem):
        return pltpu.async_copy(
            tok_hbm.at[pl.ds(sbase + c * SK_CHUNK, SK_CHUNK)], tok_v, sem)

    def fire_gather(tok_v, sr_v, sem):
        return pltpu.async_copy(skill_t.at[tok_v], sr_v, sem)

    def wait_gather(tok_v, sr_v, sem):
        pltpu.make_async_copy(skill_t.at[tok_v], sr_v, sem).wait()

    def wait_tok(c, tok_v, sem):
        pltpu.make_async_copy(
            tok_hbm.at[pl.ds(sbase + c * SK_CHUNK, SK_CHUNK)], tok_v,
            sem).wait()

    def reduce_chunk(c, sr_v):
        for s in range(SPC):
            r0 = s * SKILL_LEN
            row = c * SPC + s
            p0, p1 = [], []
            for g in range(SKILL_LEN // GRP):
                b0 = r0 + g * GRP
                a0, a1 = plsc.unpack(sr_v[b0, :],
                                     format=plsc.PackFormat.INTERLEAVED,
                                     preferred_element_type=jnp.float32)
                for j in range(1, GRP):
                    u0, u1 = plsc.unpack(sr_v[b0 + j, :],
                                         format=plsc.PackFormat.INTERLEAVED,
                                         preferred_element_type=jnp.float32)
                    a0 = a0 + u0
                    a1 = a1 + u1
                p0.append(a0)
                p1.append(a1)
            t0 = (p0[0] + p0[1]) + (p0[2] + p0[3]) + p0[4]
            t1 = (p1[0] + p1[1]) + (p1[2] + p1[3]) + p1[4]
            out_v[row, :] = plsc.pack(t0, t1,
                                      format=plsc.PackFormat.INTERLEAVED,
                                      preferred_element_type=jnp.bfloat16)

    load_tok(0, tok0, lsem0).wait()
    fire_gather(tok0, sr0, gsem0)
    load_tok(1, tok1, lsem1).wait()
    fire_gather(tok1, sr1, gsem1)

    @pl.loop(0, NCHUNK - 2, step=2)
    def _(c):
        wait_gather(tok0, sr0, gsem0)
        load_tok(c + 2, tok0, lsem0)
        reduce_chunk(c, sr0)
        wait_tok(c + 2, tok0, lsem0)
        fire_gather(tok0, sr0, gsem0)
        wait_gather(tok1, sr1, gsem1)
        load_tok(c + 3, tok1, lsem1)
        reduce_chunk(c + 1, sr1)
        wait_tok(c + 3, tok1, lsem1)
        fire_gather(tok1, sr1, gsem1)

    wait_gather(tok0, sr0, gsem0)
    reduce_chunk(NCHUNK - 2, sr0)
    wait_gather(tok1, sr1, gsem1)
    reduce_chunk(NCHUNK - 1, sr1)

    pltpu.sync_copy(out_v, out_hbm.at[pl.ds(wid * BPW, BPW), :])


@jax.jit
def _sc_skill(tok_flat, skill_t):
    bf16 = jnp.bfloat16
    i32 = jnp.int32
    scratch = [
        pltpu.VMEM((SK_CHUNK,), i32),
        pltpu.VMEM((SK_CHUNK,), i32),
        pltpu.VMEM((SK_CHUNK, 32), bf16),
        pltpu.VMEM((SK_CHUNK, 32), bf16),
        pltpu.VMEM((BPW, 32), bf16),
        pltpu.VMEM_SHARED((10000, 32), bf16),
        pltpu.SemaphoreType.DMA,
        pltpu.SemaphoreType.DMA,
        pltpu.SemaphoreType.DMA,
        pltpu.SemaphoreType.DMA,
    ]
    mesh = plsc.VectorSubcoreMesh(core_axis_name="c", subcore_axis_name="s")
    k = pl.kernel(_sc_skill_body,
                  out_type=jax.ShapeDtypeStruct((B, 32), bf16),
                  mesh=mesh, scratch_types=scratch, **_SC_PARAMS)
    return k(tok_flat, skill_t)


BB = 2048


def _mlp_body(fa, sk, lev, W1a, W1b, Wlev, b1, W2, b2, W3, b3, out):
    h = jnp.dot(fa[...], W1a[...], preferred_element_type=jnp.float32)
    h = h + jnp.dot(sk[...], W1b[...], preferred_element_type=jnp.float32)
    ids = lev[...]
    onehot = (jax.lax.broadcasted_iota(jnp.int32, (BB, 16), 1)
              == ids[:, None]).astype(jnp.bfloat16)
    h = h + jnp.dot(onehot, Wlev[...], preferred_element_type=jnp.float32)
    h = jnp.maximum(h + b1[...], 0.0)
    h = jnp.maximum(jnp.dot(h.astype(jnp.bfloat16), W2[...],
                            preferred_element_type=jnp.float32) + b2[...], 0.0)
    out[...] = jnp.dot(h.astype(jnp.bfloat16), W3[...],
                       preferred_element_type=jnp.float32) + b3[...]


@jax.jit
def _tc_mlp(fa, sk, lev, W1a, W1b, Wlev, b1, W2, b2, W3, b3):
    nb = B // BB
    full = lambda a: pl.BlockSpec(a.shape, lambda i: tuple(0 for _ in a.shape))
    return pl.pallas_call(
        _mlp_body,
        grid=(nb,),
        in_specs=[pl.BlockSpec((BB, 128), lambda i: (i, 0)),
                  pl.BlockSpec((BB, 32), lambda i: (i, 0)),
                  pl.BlockSpec((BB,), lambda i: (i,)),
                  full(W1a), full(W1b), full(Wlev), full(b1),
                  full(W2), full(b2), full(W3), full(b3)],
        out_specs=pl.BlockSpec((BB, 64), lambda i: (i, 0)),
        out_shape=jax.ShapeDtypeStruct((B, 64), jnp.float32),
    )(fa, sk, lev, W1a, W1b, Wlev, b1, W2, b2, W3, b3)


def kernel(job_id, category, location, level, skill_tokens,
           job_table, category_table, location_table, level_table, skill_table,
           W1, b1, W2, b2, W3, b3):
    bf16 = jnp.bfloat16
    fa = _sc_features(job_id, category, location,
                      job_table.astype(bf16), category_table.astype(bf16),
                      location_table.astype(bf16))
    sk = _sc_skill(skill_tokens.reshape(-1), skill_table.astype(bf16))
    W1a = W1[:128].astype(bf16)
    W1b = (W1[160:] * (1.0 / SKILL_LEN)).astype(bf16)
    Wlev = jnp.zeros((16, 256), jnp.float32).at[:11].set(
        level_table @ W1[128:160]).astype(bf16)
    return _tc_mlp(fa, sk, level, W1a, W1b, Wlev, b1,
                   W2.astype(bf16), b2, W3.astype(bf16), b3)
